# Initial kernel scaffold; baseline (speedup 1.0000x reference)
import jax
import jax.numpy as jnp
from jax.experimental import pallas as pl


def kernel(x, table):
    raise NotImplementedError("write your pallas kernel here")


# speedup vs baseline: 4.8067x; 4.8067x over previous
import functools

import jax
import jax.numpy as jnp
from jax import lax
from jax.experimental import pallas as pl
from jax.experimental.pallas import tpu as pltpu
from jax.experimental.pallas import tpu_sc as plsc

_DIM = 32
_ROWS = 16384
_COLS = 200
_B = _ROWS * _COLS
_NC, _NS = 2, 16
_NW = _NC * _NS
_BPW = _B // _NW
_CHUNK = 1024
_NCH = _BPW // _CHUNK

_mesh = plsc.VectorSubcoreMesh(core_axis_name="c", subcore_axis_name="s")


@functools.partial(
    pl.kernel,
    out_type=jax.ShapeDtypeStruct((_B, _DIM), jnp.float32),
    mesh=_mesh,
    scratch_types=[
        pltpu.VMEM((_CHUNK,), jnp.int32),
        pltpu.VMEM((_CHUNK, _DIM), jnp.float32),
        pltpu.SemaphoreType.DMA,
    ],
    compiler_params=pltpu.CompilerParams(use_tc_tiling_on_sc=False),
)
def _gather(x_hbm, table_hbm, out_hbm, idx_v, rows_v, sem):
    wid = lax.axis_index("s") * _NC + lax.axis_index("c")

    def body(g, carry):
        base = wid * _BPW + g * _CHUNK
        pltpu.sync_copy(x_hbm.at[pl.ds(base, _CHUNK)], idx_v)
        pltpu.async_copy(table_hbm.at[idx_v], rows_v, sem).wait()
        pltpu.sync_copy(rows_v, out_hbm.at[pl.ds(base, _CHUNK)])
        return carry

    lax.fori_loop(0, _NCH, body, 0)


def kernel(x, table):
    out = _gather(x.reshape(_B), table)
    return out.reshape(_ROWS, _COLS, _DIM)


# speedup vs baseline: 5.0491x; 1.0504x over previous
import functools

import jax
import jax.numpy as jnp
from jax import lax
from jax.experimental import pallas as pl
from jax.experimental.pallas import tpu as pltpu
from jax.experimental.pallas import tpu_sc as plsc

_DIM = 32
_ROWS = 16384
_COLS = 200
_B = _ROWS * _COLS
_NC, _NS = 2, 16
_NW = _NC * _NS
_BPW = _B // _NW
_CHUNK = 1024
_NCH = _BPW // _CHUNK
_NBUF = 3
_T = _NCH + _NBUF
_T_OUTER = ((_T + _NBUF - 1) // _NBUF) * _NBUF

_mesh = plsc.VectorSubcoreMesh(core_axis_name="c", subcore_axis_name="s")


@functools.partial(
    pl.kernel,
    out_type=jax.ShapeDtypeStruct((_B, _DIM), jnp.float32),
    mesh=_mesh,
    scratch_types=[
        pltpu.VMEM((_NBUF, _CHUNK), jnp.int32),
        pltpu.VMEM((_NBUF, _CHUNK, _DIM), jnp.float32),
        pltpu.SemaphoreType.DMA((_NBUF,)),
        pltpu.SemaphoreType.DMA((_NBUF,)),
        pltpu.SemaphoreType.DMA((_NBUF,)),
    ],
    compiler_params=pltpu.CompilerParams(use_tc_tiling_on_sc=False),
)
def _gather(x_hbm, table_hbm, out_hbm, idx_v, rows_v, sem_i, sem_g, sem_s):
    wid = lax.axis_index("s") * _NC + lax.axis_index("c")
    wbase = wid * _BPW

    def idx_cp(g, b):
        return pltpu.make_async_copy(
            x_hbm.at[pl.ds(wbase + g * _CHUNK, _CHUNK)], idx_v.at[b], sem_i.at[b])

    def gat_cp(b):
        return pltpu.make_async_copy(
            table_hbm.at[idx_v.at[b]], rows_v.at[b], sem_g.at[b])

    def st_cp(g, b):
        return pltpu.make_async_copy(
            rows_v.at[b], out_hbm.at[pl.ds(wbase + g * _CHUNK, _CHUNK)], sem_s.at[b])

    idx_cp(0, 0).start()
    idx_cp(1, 1).start()

    @pl.loop(0, _T_OUTER, step=_NBUF)
    def _(t):
        for j in range(_NBUF):
            g = t + j
            b = j
            bm1 = (j - 1) % _NBUF
            bp2 = (j + 2) % _NBUF

            @pl.when(jnp.logical_and(g >= _NBUF, g <= _NCH + _NBUF - 1))
            def _():
                st_cp(g - _NBUF, b).wait()

            @pl.when(g < _NCH)
            def _():
                idx_cp(g, b).wait()
                gat_cp(b).start()

            @pl.when(jnp.logical_and(g >= 1, g <= _NCH))
            def _():
                gat_cp(bm1).wait()
                st_cp(g - 1, bm1).start()

            @pl.when(g + 2 < _NCH)
            def _():
                idx_cp(g + 2, bp2).start()


def kernel(x, table):
    out = _gather(x.reshape(_B), table)
    return out.reshape(_ROWS, _COLS, _DIM)
